# Initial kernel scaffold; baseline (speedup 1.0000x reference)
import jax
import jax.numpy as jnp
from jax.experimental import pallas as pl


def kernel(data, node_mars, mus, log_scales, vids, d2vids, vrangeslow, vrangeshigh, vhbinsizes):
    raise NotImplementedError("write your pallas kernel here")


# speedup vs baseline: 4.2327x; 4.2327x over previous
import jax
import jax.numpy as jnp
from jax.experimental import pallas as pl
from jax.experimental.pallas import tpu as pltpu

_EPS = 1e-8


def _tc_body(data_ref, mus_ref, ls_ref, vlow_ref, vhigh_ref, vhb_ref, out_ref):
    npv, b = out_ref.shape
    v = pl.program_id(0)
    low = vlow_ref[v, 0]
    high = vhigh_ref[v, 0]
    hb = vhb_ref[v, 0]

    sd_row = (data_ref[pl.ds(v, 1), :] - low) * (1.0 / (high - low))

    mu = mus_ref[pl.ds(v, 1), :]
    ls = jnp.maximum(ls_ref[pl.ds(v, 1), :], -5.0)
    inv_scale = jnp.exp(-ls)
    mu1 = (mu + hb) * inv_scale
    kexp = jnp.exp((-2.0 * hb) * inv_scale)
    a1 = (1.0 + _EPS) - kexp
    a2 = kexp * (1.0 + _EPS)

    isc_c = inv_scale.reshape(npv, 1)
    mu1_c = mu1.reshape(npv, 1)
    k_c = kexp.reshape(npv, 1)
    a1_c = a1.reshape(npv, 1)
    a2_c = a2.reshape(npv, 1)

    el = jnp.exp(mu1_c - sd_row * isc_c)
    p = el + 1.0
    q = el * k_c + 1.0
    low_m = sd_row < 0.01
    high_m = sd_row > 0.99
    numer = jnp.where(low_m, 1.0,
                      jnp.where(high_m, el * a2_c + _EPS, el * a1_c + _EPS))
    denom = jnp.where(low_m, p, jnp.where(high_m, q, p * q))
    out_ref[...] = jnp.log(numer) - jnp.log(denom)


def kernel(data, node_mars, mus, log_scales, vids, d2vids, vrangeslow,
           vrangeshigh, vhbinsizes):
    nv, b = data.shape
    nn = mus.shape[0]
    npv = nn // nv
    mus2 = mus.reshape(nv, npv)
    ls2 = log_scales.reshape(nv, npv)
    return pl.pallas_call(
        _tc_body,
        grid=(nv,),
        in_specs=[
            pl.BlockSpec((nv, b), lambda v: (0, 0)),
            pl.BlockSpec((nv, npv), lambda v: (0, 0)),
            pl.BlockSpec((nv, npv), lambda v: (0, 0)),
            pl.BlockSpec(memory_space=pltpu.SMEM),
            pl.BlockSpec(memory_space=pltpu.SMEM),
            pl.BlockSpec(memory_space=pltpu.SMEM),
        ],
        out_specs=pl.BlockSpec((npv, b), lambda v: (v, 0)),
        out_shape=jax.ShapeDtypeStruct((nn, b), jnp.float32),
    )(data, mus2, ls2, vrangeslow, vrangeshigh, vhbinsizes)
